# baseline (speedup 1.0000x reference)
import functools
import math

import jax
import jax.numpy as jnp
from jax import lax
from jax.experimental import pallas as pl
from jax.experimental.pallas import tpu as pltpu
from jax.experimental.pallas import tpu_sc as plsc

H = 32
HD = 64
L = 2048
FP = 1152
TOPK = 7
NEG = -3.0e38

A_TF, A_TK, A_TN = 576, 1024, 512
B_TL, B_TF = 256, 384

_DOT = dict(preferred_element_type=jnp.float32, precision=lax.Precision.HIGHEST)


def _dft_mats():
    f = jnp.arange(FP, dtype=jnp.int32)[:, None]
    t = jnp.arange(L, dtype=jnp.int32)[None, :]
    ang = (2.0 * jnp.pi / L) * ((f * t) % L).astype(jnp.float32)
    valid = (f <= L // 2).astype(jnp.float32)
    c = jnp.cos(ang) * valid
    s = jnp.sin(ang) * valid
    wgt = jnp.where((f == 0) | (f == L // 2), 1.0, 2.0) * valid / L
    ic = (c * wgt).T
    is_ = -(s * wgt).T
    return c, s, ic, is_


def _dft_fwd_body(c_ref, s_ref, q_ref, k_ref, rr_ref, ri_ref,
                  a_acc, b_acc, c_acc, d_acc):
    kk = pl.program_id(2)
    cm, sm = c_ref[...], s_ref[...]
    qm, km = q_ref[...], k_ref[...]
    pa = jnp.dot(cm, qm, **_DOT)
    pb = jnp.dot(sm, qm, **_DOT)
    pc = jnp.dot(cm, km, **_DOT)
    pd = jnp.dot(sm, km, **_DOT)

    @pl.when(kk == 0)
    def _():
        a_acc[...] = jnp.zeros_like(a_acc)
        b_acc[...] = jnp.zeros_like(b_acc)
        c_acc[...] = jnp.zeros_like(c_acc)
        d_acc[...] = jnp.zeros_like(d_acc)

    a = a_acc[...] + pa
    b = b_acc[...] + pb
    c = c_acc[...] + pc
    d = d_acc[...] + pd
    a_acc[...] = a
    b_acc[...] = b
    c_acc[...] = c
    d_acc[...] = d

    @pl.when(kk == pl.num_programs(2) - 1)
    def _():
        rr_ref[...] = a * c + b * d
        ri_ref[...] = a * d - b * c


def _dft_fwd(cmat, smat, qmat, kmat):
    N = qmat.shape[1]
    grid = (N // A_TN, FP // A_TF, L // A_TK)
    return pl.pallas_call(
        _dft_fwd_body,
        grid=grid,
        in_specs=[
            pl.BlockSpec((A_TF, A_TK), lambda n, f, k: (f, k)),
            pl.BlockSpec((A_TF, A_TK), lambda n, f, k: (f, k)),
            pl.BlockSpec((A_TK, A_TN), lambda n, f, k: (k, n)),
            pl.BlockSpec((A_TK, A_TN), lambda n, f, k: (k, n)),
        ],
        out_specs=[
            pl.BlockSpec((A_TF, A_TN), lambda n, f, k: (f, n)),
            pl.BlockSpec((A_TF, A_TN), lambda n, f, k: (f, n)),
        ],
        out_shape=[
            jax.ShapeDtypeStruct((FP, N), jnp.float32),
            jax.ShapeDtypeStruct((FP, N), jnp.float32),
        ],
        scratch_shapes=[pltpu.VMEM((A_TF, A_TN), jnp.float32)] * 4,
    )(cmat, smat, qmat, kmat)


def _inv_topk_body(ic_ref, is_ref, rr_ref, ri_ref, w16_ref, i8_ref):
    p = pl.program_id(0)
    lb = pl.program_id(1)
    x = (jnp.dot(ic_ref[...], rr_ref[...], **_DOT) +
         jnp.dot(is_ref[...], ri_ref[...], **_DOT))

    lane = lax.broadcasted_iota(jnp.int32, (B_TL, 2 * HD), 1)
    first = lane < HD
    lglob = lax.broadcasted_iota(jnp.int32, (B_TL, 1), 0) + lb * B_TL

    ms_a, ds_a, ms_b, ds_b = [], [], [], []
    xw = x
    for _ in range(TOPK):
        xa = jnp.where(first, xw, NEG)
        ma = jnp.max(xa, axis=1, keepdims=True)
        ia = jnp.min(jnp.where(xa == ma, lane, 4096), axis=1, keepdims=True)
        xb = jnp.where(first, NEG, xw)
        mb = jnp.max(xb, axis=1, keepdims=True)
        ib = jnp.min(jnp.where(xb == mb, lane, 4096), axis=1, keepdims=True)
        ms_a.append(ma)
        ds_a.append(ia)
        ms_b.append(mb)
        ds_b.append(ib - HD)
        xw = jnp.where((lane == ia) | (lane == ib), NEG, xw)

    lane8 = lax.broadcasted_iota(jnp.int32, (B_TL, 8), 1)
    slot = lax.broadcasted_iota(jnp.int32, (B_TL, 128), 1) // 16

    def emit(ms, ds, bh):
        es = [jnp.exp(m - ms[0]) for m in ms]
        tot = es[0]
        for e in es[1:]:
            tot = tot + e
        b = bh // H
        h = bh % H
        w128 = jnp.zeros((B_TL, 128), jnp.float32)
        i8 = jnp.zeros((B_TL, 8), jnp.int32)
        for i in range(TOPK):
            lmod = lglob - ds[i]
            lmod = jnp.where(lmod < 0, lmod + L, lmod)
            gidx = (b * L + lmod) * H + h
            w128 = w128 + jnp.where(slot == i, es[i] / tot, 0.0)
            i8 = i8 + jnp.where(lane8 == i, gidx, 0)
        return w128, i8

    w128a, i8a = emit(ms_a, ds_a, 2 * p)
    w128b, i8b = emit(ms_b, ds_b, 2 * p + 1)
    w16_ref[0] = w128a
    w16_ref[1] = w128b
    i8_ref[0] = i8a
    i8_ref[1] = i8b


def _inv_topk(icmat, ismat, rr, ri):
    BH = 2 * H
    grid = (BH // 2, L // B_TL)
    return pl.pallas_call(
        _inv_topk_body,
        grid=grid,
        in_specs=[
            pl.BlockSpec((B_TL, FP), lambda p, lb: (lb, 0)),
            pl.BlockSpec((B_TL, FP), lambda p, lb: (lb, 0)),
            pl.BlockSpec((FP, 2 * HD), lambda p, lb: (0, p)),
            pl.BlockSpec((FP, 2 * HD), lambda p, lb: (0, p)),
        ],
        out_specs=[
            pl.BlockSpec((2, B_TL, 128), lambda p, lb: (p, lb, 0)),
            pl.BlockSpec((2, B_TL, 8), lambda p, lb: (p, lb, 0)),
        ],
        out_shape=[
            jax.ShapeDtypeStruct((BH, L, 128), jnp.float32),
            jax.ShapeDtypeStruct((BH, L, 8), jnp.int32),
        ],
    )(icmat, ismat, rr, ri)


SC_ROWS = 2 * H * L
SC_NW = 32
SC_RPW = SC_ROWS // SC_NW
SC_CH = 32
SC_CHUNKS = SC_RPW // SC_CH


def _sc_gather_combine(table, idx_flat, w_flat):
    mesh = plsc.VectorSubcoreMesh(core_axis_name="c", subcore_axis_name="s")

    NG = SC_CH * 8 // 128

    @functools.partial(
        pl.kernel,
        mesh=mesh,
        compiler_params=pltpu.CompilerParams(use_tc_tiling_on_sc=False),
        out_type=jax.ShapeDtypeStruct((SC_ROWS, HD), jnp.float32),
        scratch_types=[
            pltpu.VMEM((2, SC_CH * 8), jnp.int32),
            pltpu.VMEM((2, SC_CH * 128), jnp.float32),
            pltpu.VMEM((2, SC_CH * 8, HD), jnp.float32),
            pltpu.VMEM((2, SC_CH, HD), jnp.float32),
            pltpu.SemaphoreType.DMA,
            pltpu.SemaphoreType.DMA,
        ],
    )
    def run(table_hbm, idx_hbm, w_hbm, out_hbm, idx_v, w_v, rows_v, out_v,
            sem_in, sem_g):
        wid = lax.axis_index("s") * 2 + lax.axis_index("c")
        base = wid * SC_RPW

        def fetch(ci, si):
            r0 = base + ci * SC_CH
            pltpu.async_copy(idx_hbm.at[pl.ds(r0 * 8, SC_CH * 8)],
                             idx_v.at[si], sem_in)
            pltpu.async_copy(w_hbm.at[pl.ds(r0 * 128, SC_CH * 128)],
                             w_v.at[si], sem_in)

        def wait_in(si):
            pltpu.make_async_copy(idx_hbm.at[pl.ds(0, SC_CH * 8)],
                                  idx_v.at[si], sem_in).wait()
            pltpu.make_async_copy(w_hbm.at[pl.ds(0, SC_CH * 128)],
                                  w_v.at[si], sem_in).wait()

        def gather(si):
            for g in range(NG):
                pltpu.async_copy(
                    table_hbm.at[idx_v.at[si, pl.ds(g * 128, 128)]],
                    rows_v.at[si, pl.ds(g * 128, 128)], sem_g)

        def wait_g(si):
            for g in range(NG):
                pltpu.make_async_copy(
                    table_hbm.at[idx_v.at[si, pl.ds(g * 128, 128)]],
                    rows_v.at[si, pl.ds(g * 128, 128)], sem_g).wait()

        fetch(0, 0)
        wait_in(0)
        gather(0)
        fetch(1, 1)

        def pair(cc, carry):
            for sub in range(2):
                ci = 2 * cc + sub
                si = sub
                so = 1 - sub

                @pl.when(ci + 1 < SC_CHUNKS)
                def _():
                    wait_in(so)
                    gather(so)

                wait_g(si)
                for r in range(SC_CH):
                    wsp = [
                        w_v[si, pl.ds((r * 8 + j) * 16, 16)]
                        for j in range(TOPK)
                    ]
                    for c in range(HD // 16):
                        acc = wsp[0] * rows_v[si, r * 8 + 0, pl.ds(c * 16, 16)]
                        for j in range(1, TOPK):
                            acc = acc + wsp[j] * rows_v[si, r * 8 + j,
                                                        pl.ds(c * 16, 16)]
                        out_v[si, r, pl.ds(c * 16, 16)] = acc
                pltpu.sync_copy(out_v.at[si],
                                out_hbm.at[pl.ds(base + ci * SC_CH, SC_CH)])

                @pl.when(ci + 2 < SC_CHUNKS)
                def _():
                    fetch(ci + 2, si)
            return carry

        lax.fori_loop(0, SC_CHUNKS // 2, pair, 0)

    return run(table, idx_flat, w_flat)


def kernel(queries, keys, values, attn_mask):
    B, _, E = queries.shape
    cmat, smat, icmat, ismat = _dft_mats()
    qmat = jnp.concatenate([queries[b] for b in range(B)], axis=1)
    kmat = jnp.concatenate([keys[b] for b in range(B)], axis=1)
    rr, ri = _dft_fwd(cmat, smat, qmat, kmat)
    w16, i8 = _inv_topk(icmat, ismat, rr, ri)
    table = values.reshape(B * L * H, HD)
    out = _sc_gather_combine(table, i8.reshape(-1), w16.reshape(-1))
    out = out.reshape(B, H, L, HD).transpose(0, 2, 1, 3).reshape(B, L, E)
    return out


# speedup vs baseline: 1.3892x; 1.3892x over previous
import functools
import math

import jax
import jax.numpy as jnp
from jax import lax
from jax.experimental import pallas as pl
from jax.experimental.pallas import tpu as pltpu
from jax.experimental.pallas import tpu_sc as plsc

H = 32
HD = 64
L = 2048
FP = 1152
TOPK = 7
NEG = -3.0e38

A_TF, A_TK, A_TN = 576, 1024, 512
B_TL, B_TF = 256, 384

_DOT = dict(preferred_element_type=jnp.float32, precision=lax.Precision.HIGHEST)


def _dft_mats():
    f = jnp.arange(FP, dtype=jnp.int32)[:, None]
    t = jnp.arange(L, dtype=jnp.int32)[None, :]
    ang = (2.0 * jnp.pi / L) * ((f * t) % L).astype(jnp.float32)
    valid = (f <= L // 2).astype(jnp.float32)
    c = jnp.cos(ang) * valid
    s = jnp.sin(ang) * valid
    wgt = jnp.where((f == 0) | (f == L // 2), 1.0, 2.0) * valid / L
    ic = (c * wgt).T
    is_ = -(s * wgt).T
    return c, s, ic, is_


def _dft_fwd_body(c_ref, s_ref, q_ref, k_ref, rr_ref, ri_ref,
                  a_acc, b_acc, c_acc, d_acc):
    kk = pl.program_id(2)
    cm, sm = c_ref[...], s_ref[...]
    qm, km = q_ref[...], k_ref[...]
    pa = jnp.dot(cm, qm, **_DOT)
    pb = jnp.dot(sm, qm, **_DOT)
    pc = jnp.dot(cm, km, **_DOT)
    pd = jnp.dot(sm, km, **_DOT)

    @pl.when(kk == 0)
    def _():
        a_acc[...] = jnp.zeros_like(a_acc)
        b_acc[...] = jnp.zeros_like(b_acc)
        c_acc[...] = jnp.zeros_like(c_acc)
        d_acc[...] = jnp.zeros_like(d_acc)

    a = a_acc[...] + pa
    b = b_acc[...] + pb
    c = c_acc[...] + pc
    d = d_acc[...] + pd
    a_acc[...] = a
    b_acc[...] = b
    c_acc[...] = c
    d_acc[...] = d

    @pl.when(kk == pl.num_programs(2) - 1)
    def _():
        rr_ref[...] = a * c + b * d
        ri_ref[...] = a * d - b * c


def _dft_fwd(cmat, smat, qmat, kmat):
    N = qmat.shape[1]
    grid = (N // A_TN, FP // A_TF, L // A_TK)
    return pl.pallas_call(
        _dft_fwd_body,
        grid=grid,
        in_specs=[
            pl.BlockSpec((A_TF, A_TK), lambda n, f, k: (f, k)),
            pl.BlockSpec((A_TF, A_TK), lambda n, f, k: (f, k)),
            pl.BlockSpec((A_TK, A_TN), lambda n, f, k: (k, n)),
            pl.BlockSpec((A_TK, A_TN), lambda n, f, k: (k, n)),
        ],
        out_specs=[
            pl.BlockSpec((A_TF, A_TN), lambda n, f, k: (f, n)),
            pl.BlockSpec((A_TF, A_TN), lambda n, f, k: (f, n)),
        ],
        out_shape=[
            jax.ShapeDtypeStruct((FP, N), jnp.float32),
            jax.ShapeDtypeStruct((FP, N), jnp.float32),
        ],
        scratch_shapes=[pltpu.VMEM((A_TF, A_TN), jnp.float32)] * 4,
    )(cmat, smat, qmat, kmat)


def _inv_topk_body(ic_ref, is_ref, rr_ref, ri_ref, w8_ref, i8_ref):
    p = pl.program_id(0)
    lb = pl.program_id(1)
    x = (jnp.dot(ic_ref[...], rr_ref[...], **_DOT) +
         jnp.dot(is_ref[...], ri_ref[...], **_DOT))

    lane = lax.broadcasted_iota(jnp.int32, (B_TL, 2 * HD), 1)
    first = lane < HD
    lglob = lax.broadcasted_iota(jnp.int32, (B_TL, 1), 0) + lb * B_TL

    ms_a, ds_a, ms_b, ds_b = [], [], [], []
    xw = x
    for _ in range(TOPK):
        xa = jnp.where(first, xw, NEG)
        ma = jnp.max(xa, axis=1, keepdims=True)
        ia = jnp.min(jnp.where(xa == ma, lane, 4096), axis=1, keepdims=True)
        xb = jnp.where(first, NEG, xw)
        mb = jnp.max(xb, axis=1, keepdims=True)
        ib = jnp.min(jnp.where(xb == mb, lane, 4096), axis=1, keepdims=True)
        ms_a.append(ma)
        ds_a.append(ia)
        ms_b.append(mb)
        ds_b.append(ib - HD)
        xw = jnp.where((lane == ia) | (lane == ib), NEG, xw)

    lane8 = lax.broadcasted_iota(jnp.int32, (B_TL, 8), 1)

    def emit(ms, ds):
        es = [jnp.exp(m - ms[0]) for m in ms]
        tot = es[0]
        for e in es[1:]:
            tot = tot + e
        w8 = jnp.zeros((B_TL, 8), jnp.float32)
        i8 = jnp.zeros((B_TL, 8), jnp.int32)
        for i in range(TOPK):
            widx = (lglob % 32) + 64 - ds[i]
            w8 = w8 + jnp.where(lane8 == i, es[i] / tot, 0.0)
            i8 = i8 + jnp.where(lane8 == i, widx, 0)
        return w8, i8

    w8a, i8a = emit(ms_a, ds_a)
    w8b, i8b = emit(ms_b, ds_b)
    w8_ref[0] = w8a
    w8_ref[1] = w8b
    i8_ref[0] = i8a
    i8_ref[1] = i8b


def _inv_topk(icmat, ismat, rr, ri):
    BH = 2 * H
    grid = (BH // 2, L // B_TL)
    return pl.pallas_call(
        _inv_topk_body,
        grid=grid,
        in_specs=[
            pl.BlockSpec((B_TL, FP), lambda p, lb: (lb, 0)),
            pl.BlockSpec((B_TL, FP), lambda p, lb: (lb, 0)),
            pl.BlockSpec((FP, 2 * HD), lambda p, lb: (0, p)),
            pl.BlockSpec((FP, 2 * HD), lambda p, lb: (0, p)),
        ],
        out_specs=[
            pl.BlockSpec((2, B_TL, 8), lambda p, lb: (p, lb, 0)),
            pl.BlockSpec((2, B_TL, 8), lambda p, lb: (p, lb, 0)),
        ],
        out_shape=[
            jax.ShapeDtypeStruct((BH, L, 8), jnp.float32),
            jax.ShapeDtypeStruct((BH, L, 8), jnp.int32),
        ],
    )(icmat, ismat, rr, ri)


SC_ROWS = 2 * H * L
SC_NW = 32
SC_RPW = SC_ROWS // SC_NW
SC_CH = 32
SC_CHUNKS = SC_RPW // SC_CH


def _sc_gather_combine(table, idx_flat, w_flat):
    mesh = plsc.VectorSubcoreMesh(core_axis_name="c", subcore_axis_name="s")

    WIN = 96
    EXT = L + 64
    NGRP = SC_CH // 16

    @functools.partial(
        pl.kernel,
        mesh=mesh,
        compiler_params=pltpu.CompilerParams(
            use_tc_tiling_on_sc=False, needs_layout_passes=False),
        out_type=jax.ShapeDtypeStruct((SC_ROWS * HD,), jnp.float32),
        scratch_types=[
            [pltpu.VMEM((SC_CH * 8,), jnp.int32)] * 2,
            [pltpu.VMEM((SC_CH * 8,), jnp.float32)] * 2,
            [pltpu.VMEM((WIN * HD,), jnp.float32)] * 2,
            [pltpu.VMEM((SC_CH * HD,), jnp.float32)] * 2,
            pltpu.SemaphoreType.DMA,
        ],
    )
    def run(vext_hbm, idx_hbm, w_hbm, out_hbm, idx_v, w_v, win_v,
            out_v, sem_in):
        wid = lax.axis_index("s") * 2 + lax.axis_index("c")
        base = wid * SC_RPW
        iota = lax.iota(jnp.int32, 16)

        def fetch(ci, si):
            r0 = base + ci * SC_CH
            bh = r0 // L
            l0 = r0 % L
            pltpu.async_copy(idx_hbm.at[pl.ds(r0 * 8, SC_CH * 8)],
                             idx_v[si], sem_in)
            pltpu.async_copy(w_hbm.at[pl.ds(r0 * 8, SC_CH * 8)],
                             w_v[si], sem_in)
            pltpu.async_copy(vext_hbm.at[pl.ds((bh * EXT + l0) * HD,
                                               WIN * HD)],
                             win_v[si], sem_in)

        def wait_in(si):
            pltpu.make_async_copy(idx_hbm.at[pl.ds(0, SC_CH * 8)],
                                  idx_v[si], sem_in).wait()
            pltpu.make_async_copy(w_hbm.at[pl.ds(0, SC_CH * 8)],
                                  w_v[si], sem_in).wait()
            pltpu.make_async_copy(vext_hbm.at[pl.ds(0, WIN * HD)],
                                  win_v[si], sem_in).wait()

        fetch(0, 0)
        fetch(1, 1)

        def pair(cc, carry):
            for sub in range(2):
                ci = 2 * cc + sub
                si = sub

                wait_in(si)
                for g in range(NGRP):
                    pos0 = iota * 8 + g * 16 * 8
                    wvec, bvec = [], []
                    for j in range(TOPK):
                        wvec.append(plsc.load_gather(w_v[si], [pos0 + j]))
                        wdx = plsc.load_gather(idx_v[si], [pos0 + j])
                        bvec.append(wdx * HD)
                    ob = g * 16 * HD + iota * HD

                    def col(c, carry2):
                        acc = wvec[0] * plsc.load_gather(
                            win_v[si], [bvec[0] + c])
                        for j in range(1, TOPK):
                            acc = acc + wvec[j] * plsc.load_gather(
                                win_v[si], [bvec[j] + c])
                        plsc.store_scatter(out_v[si], [ob + c], acc)
                        return carry2

                    lax.fori_loop(0, HD, col, 0)
                pltpu.sync_copy(
                    out_v[si],
                    out_hbm.at[pl.ds((base + ci * SC_CH) * HD, SC_CH * HD)])

                @pl.when(ci + 2 < SC_CHUNKS)
                def _():
                    fetch(ci + 2, si)
            return carry

        lax.fori_loop(0, SC_CHUNKS // 2, pair, 0)

    return run(table, idx_flat, w_flat)


def kernel(queries, keys, values, attn_mask):
    B, _, E = queries.shape
    cmat, smat, icmat, ismat = _dft_mats()
    qmat = jnp.concatenate([queries[b] for b in range(B)], axis=1)
    kmat = jnp.concatenate([keys[b] for b in range(B)], axis=1)
    rr, ri = _dft_fwd(cmat, smat, qmat, kmat)
    w8, i8 = _inv_topk(icmat, ismat, rr, ri)
    v2 = values.reshape(B, L, H, HD).transpose(0, 2, 1, 3).reshape(B * H, L, HD)
    vext = jnp.concatenate([v2[:, L - 64:], v2], axis=1)
    vext = vext.reshape(B * H * (L + 64) * HD)
    out = _sc_gather_combine(vext, i8.reshape(-1), w8.reshape(-1))
    out = out.reshape(B, H, L, HD).transpose(0, 2, 1, 3).reshape(B, L, E)
    return out


# speedup vs baseline: 1.3926x; 1.0024x over previous
import functools
import math

import jax
import jax.numpy as jnp
from jax import lax
from jax.experimental import pallas as pl
from jax.experimental.pallas import tpu as pltpu
from jax.experimental.pallas import tpu_sc as plsc

H = 32
HD = 64
L = 2048
FP = 1152
TOPK = 7
NEG = -3.0e38

A_TF, A_TK, A_TN = 576, 1024, 512
B_TL, B_TF = 256, 384

_DOT = dict(preferred_element_type=jnp.float32, precision=lax.Precision.HIGHEST)


def _dft_mats():
    f = jnp.arange(FP, dtype=jnp.int32)[:, None]
    t = jnp.arange(L, dtype=jnp.int32)[None, :]
    ang = (2.0 * jnp.pi / L) * ((f * t) % L).astype(jnp.float32)
    valid = (f <= L // 2).astype(jnp.float32)
    c = jnp.cos(ang) * valid
    s = jnp.sin(ang) * valid
    wgt = jnp.where((f == 0) | (f == L // 2), 1.0, 2.0) * valid / L
    ic = (c * wgt).T
    is_ = -(s * wgt).T
    return c, s, ic, is_


def _dft_fwd_body(c_ref, s_ref, q_ref, k_ref, rr_ref, ri_ref,
                  a_acc, b_acc, c_acc, d_acc):
    kk = pl.program_id(2)
    cm, sm = c_ref[...], s_ref[...]
    qm, km = q_ref[...], k_ref[...]
    pa = jnp.dot(cm, qm, **_DOT)
    pb = jnp.dot(sm, qm, **_DOT)
    pc = jnp.dot(cm, km, **_DOT)
    pd = jnp.dot(sm, km, **_DOT)

    @pl.when(kk == 0)
    def _():
        a_acc[...] = jnp.zeros_like(a_acc)
        b_acc[...] = jnp.zeros_like(b_acc)
        c_acc[...] = jnp.zeros_like(c_acc)
        d_acc[...] = jnp.zeros_like(d_acc)

    a = a_acc[...] + pa
    b = b_acc[...] + pb
    c = c_acc[...] + pc
    d = d_acc[...] + pd
    a_acc[...] = a
    b_acc[...] = b
    c_acc[...] = c
    d_acc[...] = d

    @pl.when(kk == pl.num_programs(2) - 1)
    def _():
        rr_ref[...] = a * c + b * d
        ri_ref[...] = a * d - b * c


def _dft_fwd(cmat, smat, qmat, kmat):
    N = qmat.shape[1]
    grid = (N // A_TN, FP // A_TF, L // A_TK)
    return pl.pallas_call(
        _dft_fwd_body,
        grid=grid,
        in_specs=[
            pl.BlockSpec((A_TF, A_TK), lambda n, f, k: (f, k)),
            pl.BlockSpec((A_TF, A_TK), lambda n, f, k: (f, k)),
            pl.BlockSpec((A_TK, A_TN), lambda n, f, k: (k, n)),
            pl.BlockSpec((A_TK, A_TN), lambda n, f, k: (k, n)),
        ],
        out_specs=[
            pl.BlockSpec((A_TF, A_TN), lambda n, f, k: (f, n)),
            pl.BlockSpec((A_TF, A_TN), lambda n, f, k: (f, n)),
        ],
        out_shape=[
            jax.ShapeDtypeStruct((FP, N), jnp.float32),
            jax.ShapeDtypeStruct((FP, N), jnp.float32),
        ],
        scratch_shapes=[pltpu.VMEM((A_TF, A_TN), jnp.float32)] * 4,
    )(cmat, smat, qmat, kmat)


def _inv_topk_body(ic_ref, is_ref, rr_ref, ri_ref, w8_ref, i8_ref):
    p = pl.program_id(0)
    lb = pl.program_id(1)
    x = (jnp.dot(ic_ref[...], rr_ref[...], **_DOT) +
         jnp.dot(is_ref[...], ri_ref[...], **_DOT))

    lane = lax.broadcasted_iota(jnp.int32, (B_TL, 2 * HD), 1)
    first = lane < HD
    lglob = lax.broadcasted_iota(jnp.int32, (B_TL, 1), 0) + lb * B_TL

    ms_a, ds_a, ms_b, ds_b = [], [], [], []
    xw = x
    for _ in range(TOPK):
        xa = jnp.where(first, xw, NEG)
        ma = jnp.max(xa, axis=1, keepdims=True)
        ia = jnp.min(jnp.where(xa == ma, lane, 4096), axis=1, keepdims=True)
        xb = jnp.where(first, NEG, xw)
        mb = jnp.max(xb, axis=1, keepdims=True)
        ib = jnp.min(jnp.where(xb == mb, lane, 4096), axis=1, keepdims=True)
        ms_a.append(ma)
        ds_a.append(ia)
        ms_b.append(mb)
        ds_b.append(ib - HD)
        xw = jnp.where((lane == ia) | (lane == ib), NEG, xw)

    lane8 = lax.broadcasted_iota(jnp.int32, (B_TL, 8), 1)

    def emit(ms, ds):
        es = [jnp.exp(m - ms[0]) for m in ms]
        tot = es[0]
        for e in es[1:]:
            tot = tot + e
        w8 = jnp.zeros((B_TL, 8), jnp.float32)
        i8 = jnp.zeros((B_TL, 8), jnp.int32)
        for i in range(TOPK):
            widx = (lglob % 32) + 64 - ds[i]
            w8 = w8 + jnp.where(lane8 == i, es[i] / tot, 0.0)
            i8 = i8 + jnp.where(lane8 == i, widx, 0)
        return w8, i8

    w8a, i8a = emit(ms_a, ds_a)
    w8b, i8b = emit(ms_b, ds_b)
    w8_ref[0] = w8a
    w8_ref[1] = w8b
    i8_ref[0] = i8a
    i8_ref[1] = i8b


def _inv_topk(icmat, ismat, rr, ri):
    BH = 2 * H
    grid = (BH // 2, L // B_TL)
    return pl.pallas_call(
        _inv_topk_body,
        grid=grid,
        in_specs=[
            pl.BlockSpec((B_TL, FP), lambda p, lb: (lb, 0)),
            pl.BlockSpec((B_TL, FP), lambda p, lb: (lb, 0)),
            pl.BlockSpec((FP, 2 * HD), lambda p, lb: (0, p)),
            pl.BlockSpec((FP, 2 * HD), lambda p, lb: (0, p)),
        ],
        out_specs=[
            pl.BlockSpec((2, B_TL, 8), lambda p, lb: (p, lb, 0)),
            pl.BlockSpec((2, B_TL, 8), lambda p, lb: (p, lb, 0)),
        ],
        out_shape=[
            jax.ShapeDtypeStruct((BH, L, 8), jnp.float32),
            jax.ShapeDtypeStruct((BH, L, 8), jnp.int32),
        ],
    )(icmat, ismat, rr, ri)


SC_ROWS = 2 * H * L
SC_NW = 32
SC_RPW = SC_ROWS // SC_NW
SC_CH = 32
SC_CHUNKS = SC_RPW // SC_CH


def _sc_gather_combine(table, idx_flat, w_flat):
    mesh = plsc.VectorSubcoreMesh(core_axis_name="c", subcore_axis_name="s")

    WIN = 96
    EXT = L + 64
    NGRP = SC_CH // 16

    @functools.partial(
        pl.kernel,
        mesh=mesh,
        compiler_params=pltpu.CompilerParams(
            use_tc_tiling_on_sc=False, needs_layout_passes=False),
        out_type=jax.ShapeDtypeStruct((SC_ROWS * HD,), jnp.float32),
        scratch_types=[
            [pltpu.VMEM((SC_CH * 8,), jnp.int32)] * 2,
            [pltpu.VMEM((SC_CH * 8,), jnp.float32)] * 2,
            [pltpu.VMEM((WIN * HD,), jnp.float32)] * 2,
            [pltpu.VMEM((SC_CH * HD,), jnp.float32)] * 2,
            pltpu.SemaphoreType.DMA,
        ],
    )
    def run(vext_hbm, idx_hbm, w_hbm, out_hbm, idx_v, w_v, win_v,
            out_v, sem_in):
        wid = lax.axis_index("s") * 2 + lax.axis_index("c")
        base = wid * SC_RPW
        iota = lax.iota(jnp.int32, 16)

        def fetch(ci, si):
            r0 = base + ci * SC_CH
            bh = r0 // L
            l0 = r0 % L
            pltpu.async_copy(idx_hbm.at[pl.ds(r0 * 8, SC_CH * 8)],
                             idx_v[si], sem_in)
            pltpu.async_copy(w_hbm.at[pl.ds(r0 * 8, SC_CH * 8)],
                             w_v[si], sem_in)
            pltpu.async_copy(vext_hbm.at[pl.ds((bh * EXT + l0) * HD,
                                               WIN * HD)],
                             win_v[si], sem_in)

        def wait_in(si):
            pltpu.make_async_copy(idx_hbm.at[pl.ds(0, SC_CH * 8)],
                                  idx_v[si], sem_in).wait()
            pltpu.make_async_copy(w_hbm.at[pl.ds(0, SC_CH * 8)],
                                  w_v[si], sem_in).wait()
            pltpu.make_async_copy(vext_hbm.at[pl.ds(0, WIN * HD)],
                                  win_v[si], sem_in).wait()

        fetch(0, 0)
        fetch(1, 1)

        def pair(cc, carry):
            for sub in range(2):
                ci = 2 * cc + sub
                si = sub

                wait_in(si)
                for g in range(NGRP):
                    pos0 = iota * 8 + g * 16 * 8
                    wvec, bvec = [], []
                    for j in range(TOPK):
                        wvec.append(plsc.load_gather(w_v[si], [pos0 + j]))
                        wdx = plsc.load_gather(idx_v[si], [pos0 + j])
                        bvec.append(wdx * HD)
                    ob = g * 16 * HD + iota * HD

                    def col(c4, carry2):
                        for u in range(4):
                            c = c4 * 4 + u
                            gs = [plsc.load_gather(win_v[si], [bvec[j] + c])
                                  for j in range(TOPK)]
                            acc = wvec[0] * gs[0]
                            for j in range(1, TOPK):
                                acc = acc + wvec[j] * gs[j]
                            plsc.store_scatter(out_v[si], [ob + c], acc)
                        return carry2

                    lax.fori_loop(0, HD // 4, col, 0)
                pltpu.sync_copy(
                    out_v[si],
                    out_hbm.at[pl.ds((base + ci * SC_CH) * HD, SC_CH * HD)])

                @pl.when(ci + 2 < SC_CHUNKS)
                def _():
                    fetch(ci + 2, si)
            return carry

        lax.fori_loop(0, SC_CHUNKS // 2, pair, 0)

    return run(table, idx_flat, w_flat)


def kernel(queries, keys, values, attn_mask):
    B, _, E = queries.shape
    cmat, smat, icmat, ismat = _dft_mats()
    qmat = jnp.concatenate([queries[b] for b in range(B)], axis=1)
    kmat = jnp.concatenate([keys[b] for b in range(B)], axis=1)
    rr, ri = _dft_fwd(cmat, smat, qmat, kmat)
    w8, i8 = _inv_topk(icmat, ismat, rr, ri)
    v2 = values.reshape(B, L, H, HD).transpose(0, 2, 1, 3).reshape(B * H, L, HD)
    vext = jnp.concatenate([v2[:, L - 64:], v2], axis=1)
    vext = vext.reshape(B * H * (L + 64) * HD)
    out = _sc_gather_combine(vext, i8.reshape(-1), w8.reshape(-1))
    out = out.reshape(B, H, L, HD).transpose(0, 2, 1, 3).reshape(B, L, E)
    return out


# speedup vs baseline: 1.8587x; 1.3347x over previous
import functools
import math

import jax
import jax.numpy as jnp
from jax import lax
from jax.experimental import pallas as pl
from jax.experimental.pallas import tpu as pltpu
from jax.experimental.pallas import tpu_sc as plsc

H = 32
HD = 64
L = 2048
FP = 1152
TOPK = 7
NEG = -3.0e38

A_TF, A_TK, A_TN = 576, 1024, 512
B_TL, B_TF = 256, 384

_DOT = dict(preferred_element_type=jnp.float32, precision=lax.Precision.HIGHEST)


def _dft_mats():
    f = jnp.arange(FP, dtype=jnp.int32)[:, None]
    t = jnp.arange(L, dtype=jnp.int32)[None, :]
    ang = (2.0 * jnp.pi / L) * ((f * t) % L).astype(jnp.float32)
    valid = (f <= L // 2).astype(jnp.float32)
    c = jnp.cos(ang) * valid
    s = jnp.sin(ang) * valid
    wgt = jnp.where((f == 0) | (f == L // 2), 1.0, 2.0) * valid / L
    ic = (c * wgt).T
    is_ = -(s * wgt).T
    return c, s, ic, is_


def _dft_fwd_body(c_ref, s_ref, q_ref, k_ref, rr_ref, ri_ref,
                  a_acc, b_acc, c_acc, d_acc):
    kk = pl.program_id(2)
    cm, sm = c_ref[...], s_ref[...]
    qm, km = q_ref[...], k_ref[...]
    pa = jnp.dot(cm, qm, **_DOT)
    pb = jnp.dot(sm, qm, **_DOT)
    pc = jnp.dot(cm, km, **_DOT)
    pd = jnp.dot(sm, km, **_DOT)

    @pl.when(kk == 0)
    def _():
        a_acc[...] = jnp.zeros_like(a_acc)
        b_acc[...] = jnp.zeros_like(b_acc)
        c_acc[...] = jnp.zeros_like(c_acc)
        d_acc[...] = jnp.zeros_like(d_acc)

    a = a_acc[...] + pa
    b = b_acc[...] + pb
    c = c_acc[...] + pc
    d = d_acc[...] + pd
    a_acc[...] = a
    b_acc[...] = b
    c_acc[...] = c
    d_acc[...] = d

    @pl.when(kk == pl.num_programs(2) - 1)
    def _():
        rr_ref[...] = a * c + b * d
        ri_ref[...] = a * d - b * c


def _dft_fwd(cmat, smat, qmat, kmat):
    N = qmat.shape[1]
    grid = (N // A_TN, FP // A_TF, L // A_TK)
    return pl.pallas_call(
        _dft_fwd_body,
        grid=grid,
        in_specs=[
            pl.BlockSpec((A_TF, A_TK), lambda n, f, k: (f, k)),
            pl.BlockSpec((A_TF, A_TK), lambda n, f, k: (f, k)),
            pl.BlockSpec((A_TK, A_TN), lambda n, f, k: (k, n)),
            pl.BlockSpec((A_TK, A_TN), lambda n, f, k: (k, n)),
        ],
        out_specs=[
            pl.BlockSpec((A_TF, A_TN), lambda n, f, k: (f, n)),
            pl.BlockSpec((A_TF, A_TN), lambda n, f, k: (f, n)),
        ],
        out_shape=[
            jax.ShapeDtypeStruct((FP, N), jnp.float32),
            jax.ShapeDtypeStruct((FP, N), jnp.float32),
        ],
        scratch_shapes=[pltpu.VMEM((A_TF, A_TN), jnp.float32)] * 4,
    )(cmat, smat, qmat, kmat)


def _inv_topk_body(ic_ref, is_ref, rr_ref, ri_ref, w8_ref, i8_ref):
    p = pl.program_id(0)
    lb = pl.program_id(1)
    x = (jnp.dot(ic_ref[...], rr_ref[...], **_DOT) +
         jnp.dot(is_ref[...], ri_ref[...], **_DOT))

    lane = lax.broadcasted_iota(jnp.int32, (B_TL, 2 * HD), 1)
    first = lane < HD
    lglob = lax.broadcasted_iota(jnp.int32, (B_TL, 1), 0) + lb * B_TL

    ms_a, ds_a, ms_b, ds_b = [], [], [], []
    xw = x
    for _ in range(TOPK):
        xa = jnp.where(first, xw, NEG)
        ma = jnp.max(xa, axis=1, keepdims=True)
        ia = jnp.min(jnp.where(xa == ma, lane, 4096), axis=1, keepdims=True)
        xb = jnp.where(first, NEG, xw)
        mb = jnp.max(xb, axis=1, keepdims=True)
        ib = jnp.min(jnp.where(xb == mb, lane, 4096), axis=1, keepdims=True)
        ms_a.append(ma)
        ds_a.append(ia)
        ms_b.append(mb)
        ds_b.append(ib - HD)
        xw = jnp.where((lane == ia) | (lane == ib), NEG, xw)

    lane8 = lax.broadcasted_iota(jnp.int32, (B_TL, 8), 1)

    def emit(ms, ds):
        es = [jnp.exp(m - ms[0]) for m in ms]
        tot = es[0]
        for e in es[1:]:
            tot = tot + e
        w8 = jnp.zeros((B_TL, 8), jnp.float32)
        i8 = jnp.zeros((B_TL, 8), jnp.int32)
        for i in range(TOPK):
            widx = (lglob % 32) + 64 - ds[i]
            w8 = w8 + jnp.where(lane8 == i, es[i] / tot, 0.0)
            i8 = i8 + jnp.where(lane8 == i, widx, 0)
        return w8, i8

    w8a, i8a = emit(ms_a, ds_a)
    w8b, i8b = emit(ms_b, ds_b)
    w8_ref[0] = w8a
    w8_ref[1] = w8b
    i8_ref[0] = i8a
    i8_ref[1] = i8b


def _inv_topk(icmat, ismat, rr, ri):
    BH = 2 * H
    grid = (BH // 2, L // B_TL)
    return pl.pallas_call(
        _inv_topk_body,
        grid=grid,
        in_specs=[
            pl.BlockSpec((B_TL, FP), lambda p, lb: (lb, 0)),
            pl.BlockSpec((B_TL, FP), lambda p, lb: (lb, 0)),
            pl.BlockSpec((FP, 2 * HD), lambda p, lb: (0, p)),
            pl.BlockSpec((FP, 2 * HD), lambda p, lb: (0, p)),
        ],
        out_specs=[
            pl.BlockSpec((2, B_TL, 8), lambda p, lb: (p, lb, 0)),
            pl.BlockSpec((2, B_TL, 8), lambda p, lb: (p, lb, 0)),
        ],
        out_shape=[
            jax.ShapeDtypeStruct((BH, L, 8), jnp.float32),
            jax.ShapeDtypeStruct((BH, L, 8), jnp.int32),
        ],
    )(icmat, ismat, rr, ri)


SC_ROWS = 2 * H * L
SC_NW = 32
SC_RPW = SC_ROWS // SC_NW
SC_CH = 32
SC_CHUNKS = SC_RPW // SC_CH


def _sc_gather_combine(table, idx_flat, w_flat):
    mesh = plsc.VectorSubcoreMesh(core_axis_name="c", subcore_axis_name="s")

    WIN = 96
    EXT = L + 64
    NGRP = SC_CH // 16
    VR = HD + 1

    @functools.partial(
        pl.kernel,
        mesh=mesh,
        compiler_params=pltpu.CompilerParams(
            use_tc_tiling_on_sc=False, needs_layout_passes=False),
        out_type=jax.ShapeDtypeStruct((SC_ROWS * HD,), jnp.float32),
        scratch_types=[
            [pltpu.VMEM((SC_CH * 8,), jnp.int32)] * 2,
            [pltpu.VMEM((SC_CH * 8,), jnp.float32)] * 2,
            [pltpu.VMEM((WIN * VR,), jnp.float32)] * 2,
            [pltpu.VMEM((SC_CH * HD,), jnp.float32)] * 2,
            pltpu.SemaphoreType.DMA,
        ],
    )
    def run(vext_hbm, idx_hbm, w_hbm, out_hbm, idx_v, w_v, win_v,
            out_v, sem_in):
        wid = lax.axis_index("s") * 2 + lax.axis_index("c")
        base = wid * SC_RPW
        iota = lax.iota(jnp.int32, 16)

        def fetch(ci, si):
            r0 = base + ci * SC_CH
            bh = r0 // L
            l0 = r0 % L
            pltpu.async_copy(idx_hbm.at[pl.ds(r0 * 8, SC_CH * 8)],
                             idx_v[si], sem_in)
            pltpu.async_copy(w_hbm.at[pl.ds(r0 * 8, SC_CH * 8)],
                             w_v[si], sem_in)
            pltpu.async_copy(vext_hbm.at[pl.ds((bh * EXT + l0) * VR,
                                               WIN * VR)],
                             win_v[si], sem_in)

        def wait_in(si):
            pltpu.make_async_copy(idx_hbm.at[pl.ds(0, SC_CH * 8)],
                                  idx_v[si], sem_in).wait()
            pltpu.make_async_copy(w_hbm.at[pl.ds(0, SC_CH * 8)],
                                  w_v[si], sem_in).wait()
            pltpu.make_async_copy(vext_hbm.at[pl.ds(0, WIN * VR)],
                                  win_v[si], sem_in).wait()

        fetch(0, 0)
        fetch(1, 1)

        def pair(cc, carry):
            for sub in range(2):
                ci = 2 * cc + sub
                si = sub

                wait_in(si)
                for g in range(NGRP):
                    pos0 = iota * 8 + g * 16 * 8
                    wvec, bvec = [], []
                    for j in range(TOPK):
                        wvec.append(plsc.load_gather(w_v[si], [pos0 + j]))
                        wdx = plsc.load_gather(idx_v[si], [pos0 + j])
                        bvec.append(wdx * VR)
                    ob = g * 16 * HD + iota * HD

                    def col(c4, carry2):
                        for u in range(4):
                            c = c4 * 4 + u
                            gs = [plsc.load_gather(win_v[si], [bvec[j] + c])
                                  for j in range(TOPK)]
                            acc = wvec[0] * gs[0]
                            for j in range(1, TOPK):
                                acc = acc + wvec[j] * gs[j]
                            plsc.store_scatter(out_v[si], [ob + c], acc)
                        return carry2

                    lax.fori_loop(0, HD // 4, col, 0)
                pltpu.sync_copy(
                    out_v[si],
                    out_hbm.at[pl.ds((base + ci * SC_CH) * HD, SC_CH * HD)])

                @pl.when(ci + 2 < SC_CHUNKS)
                def _():
                    fetch(ci + 2, si)
            return carry

        lax.fori_loop(0, SC_CHUNKS // 2, pair, 0)

    return run(table, idx_flat, w_flat)


def kernel(queries, keys, values, attn_mask):
    B, _, E = queries.shape
    cmat, smat, icmat, ismat = _dft_mats()
    qmat = jnp.concatenate([queries[b] for b in range(B)], axis=1)
    kmat = jnp.concatenate([keys[b] for b in range(B)], axis=1)
    rr, ri = _dft_fwd(cmat, smat, qmat, kmat)
    w8, i8 = _inv_topk(icmat, ismat, rr, ri)
    v2 = values.reshape(B, L, H, HD).transpose(0, 2, 1, 3).reshape(B * H, L, HD)
    vext = jnp.concatenate([v2[:, L - 64:], v2], axis=1)
    vext = jnp.pad(vext, ((0, 0), (0, 0), (0, 1)))
    vext = vext.reshape(B * H * (L + 64) * (HD + 1))
    out = _sc_gather_combine(vext, i8.reshape(-1), w8.reshape(-1))
    out = out.reshape(B, H, L, HD).transpose(0, 2, 1, 3).reshape(B, L, E)
    return out
